# Initial kernel scaffold; baseline (speedup 1.0000x reference)
import jax
import jax.numpy as jnp
from jax.experimental import pallas as pl


def kernel(tokens, tok_emb, pos_emb):
    raise NotImplementedError("write your pallas kernel here")


# speedup vs baseline: 2.3783x; 2.3783x over previous
import functools

import jax
import jax.numpy as jnp
from jax import lax
from jax.experimental import pallas as pl
from jax.experimental.pallas import tpu as pltpu, tpu_sc as plsc

D_MODEL = 64
SEQ_LEN = 1024
NUM_CORES = 2
NUM_SUBCORES = 16
NUM_WORKERS = NUM_CORES * NUM_SUBCORES
LANES = 16
CHUNK = 128
CHUNKS_PER_SEQ = SEQ_LEN // CHUNK


def _body(tok_hbm, te_hbm, pe_hbm, out_hbm,
          toks_v, pos_v, bt, bp, carry_v, sem_t, sem_p, seq_per_worker):
    wid = lax.axis_index("s") * NUM_CORES + lax.axis_index("c")

    def per_seq(i, _):
        s = wid * seq_per_worker + i
        base_tok = s * SEQ_LEN
        pltpu.sync_copy(tok_hbm.at[pl.ds(base_tok, SEQ_LEN)], toks_v)

        carry_v[...] = jnp.zeros((LANES,), jnp.int32)

        def pos_grp(g, _):
            t16 = toks_v[pl.ds(g * LANES, LANES)]
            m = jnp.minimum(t16, 1)
            cs = plsc.cumsum(m)
            carry = carry_v[...]
            pos_v[pl.ds(g * LANES, LANES)] = (cs + carry) * m
            carry_v[...] = carry + lax.reduce_sum(m, axes=(0,))
            return 0

        lax.fori_loop(0, SEQ_LEN // LANES, pos_grp, 0)

        def chunk(c, _):
            base = c * CHUNK
            cp_t = pltpu.async_copy(
                te_hbm.at[toks_v.at[pl.ds(base, CHUNK)]], bt, sem_t)
            cp_p = pltpu.async_copy(
                pe_hbm.at[pos_v.at[pl.ds(base, CHUNK)]], bp, sem_p)
            cp_t.wait()
            cp_p.wait()

            def add_row(r, _):
                for j in range(D_MODEL // LANES):
                    sl = pl.ds(j * LANES, LANES)
                    bt[r, sl] = bt[r, sl] + bp[r, sl]
                return 0

            lax.fori_loop(0, CHUNK, add_row, 0)
            pltpu.sync_copy(bt, out_hbm.at[pl.ds(base_tok + base, CHUNK)])
            return 0

        lax.fori_loop(0, CHUNKS_PER_SEQ, chunk, 0)
        return 0

    lax.fori_loop(0, seq_per_worker, per_seq, 0)


def kernel(tokens, tok_emb, pos_emb):
    B, K, L = tokens.shape
    n_seq = B * K
    assert L == SEQ_LEN and n_seq % NUM_WORKERS == 0
    seq_per_worker = n_seq // NUM_WORKERS

    flat = tokens.reshape(n_seq * L).astype(jnp.int32)

    run = functools.partial(
        pl.kernel,
        out_type=jax.ShapeDtypeStruct((n_seq * L, D_MODEL), jnp.float32),
        mesh=plsc.VectorSubcoreMesh(core_axis_name="c", subcore_axis_name="s",
                                    num_cores=NUM_CORES,
                                    num_subcores=NUM_SUBCORES),
        scratch_types=[
            pltpu.VMEM((SEQ_LEN,), jnp.int32),
            pltpu.VMEM((SEQ_LEN,), jnp.int32),
            pltpu.VMEM((CHUNK, D_MODEL), jnp.float32),
            pltpu.VMEM((CHUNK, D_MODEL), jnp.float32),
            pltpu.VMEM((LANES,), jnp.int32),
            pltpu.SemaphoreType.DMA,
            pltpu.SemaphoreType.DMA,
        ],
        compiler_params=pltpu.CompilerParams(use_tc_tiling_on_sc=False,
                                             needs_layout_passes=False),
    )(functools.partial(_body, seq_per_worker=seq_per_worker))

    out = run(flat, tok_emb.astype(jnp.float32), pos_emb.astype(jnp.float32))
    return out.reshape(B, K, L, D_MODEL)
